# Initial kernel scaffold; baseline (speedup 1.0000x reference)
import jax
import jax.numpy as jnp
from jax.experimental import pallas as pl


def kernel(psi):
    raise NotImplementedError("write your pallas kernel here")


# speedup vs baseline: 616.6461x; 616.6461x over previous
import functools

import jax
import jax.numpy as jnp
from jax import lax
from jax.experimental import pallas as pl
from jax.experimental.pallas import tpu as pltpu
from jax.experimental.pallas import tpu_sc as plsc

N_IN = 16777216
N_OUT = 2 * N_IN
NUM_CORES = 2
NUM_SUBCORES = 16
NW = NUM_CORES * NUM_SUBCORES
S = N_IN // NW
C = 32768
NCHUNK = S // C

_mesh = plsc.VectorSubcoreMesh(
    core_axis_name="c", subcore_axis_name="s", num_cores=NUM_CORES)


@functools.partial(
    pl.kernel,
    mesh=_mesh,
    out_type=jax.ShapeDtypeStruct((N_OUT,), jnp.float32),
    scratch_types=[
        pltpu.VMEM((C,), jnp.float32),
        pltpu.VMEM((C,), jnp.float32),
    ],
)
def _add_ancilla(psi_hbm, out_hbm, buf, zbuf):
    wid = lax.axis_index("s") * NUM_CORES + lax.axis_index("c")
    base = wid * S

    zero16 = jnp.zeros((16,), jnp.float32)

    def zero_body(i, _):
        for u in range(8):
            zbuf[pl.ds((i * 8 + u) * 16, 16)] = zero16
        return 0

    lax.fori_loop(0, C // (16 * 8), zero_body, 0)

    def body(i, _):
        off = base + i * C
        pltpu.sync_copy(psi_hbm.at[pl.ds(off, C)], buf)
        pltpu.sync_copy(buf, out_hbm.at[pl.ds(off, C)])
        pltpu.sync_copy(zbuf, out_hbm.at[pl.ds(N_IN + off, C)])
        return 0

    lax.fori_loop(0, NCHUNK, body, 0)


def kernel(psi):
    return _add_ancilla(psi)
